# baseline (device time: 52386 ns/iter reference)
import jax
import jax.numpy as jnp
from jax import lax
from jax.experimental import pallas as pl
from jax.experimental.pallas import tpu as pltpu

N_DEV = 16


def kernel(x, w_mat, scale_x, scale_w):
    m_total, k_loc = x.shape
    k_total, n_out = w_mat.shape
    m_per = m_total // N_DEV

    my_pos = lax.axis_index("i")
    steps = jnp.arange(N_DEV, dtype=jnp.int32)
    wtab = jnp.remainder(my_pos.astype(jnp.int32) - steps, N_DEV)
    wtab = jnp.concatenate([wtab, wtab[-1:]])

    def body(wtab_ref, x_ref, w_ref, sx_ref, sw_ref, out_ref,
             comm_ref, wbf_ref, acc_ref, send_sems, recv_sems):
        s = pl.program_id(0)
        my = lax.axis_index("i")
        dims = (((1,), (0,)), ((), ()))

        @pl.when(s == 0)
        def _():
            barrier = pltpu.get_barrier_semaphore()
            for d in range(1, N_DEV):
                t = lax.rem(my + d, N_DEV)
                pl.semaphore_signal(barrier, inc=1, device_id=(t,),
                                    device_id_type=pl.DeviceIdType.MESH)
            pl.semaphore_wait(barrier, N_DEV - 1)

            for d in range(1, N_DEV):
                t = lax.rem(my + d, N_DEV)
                pltpu.make_async_remote_copy(
                    src_ref=x_ref.at[pl.ds(t * m_per, m_per), :],
                    dst_ref=comm_ref.at[d],
                    send_sem=send_sems.at[d],
                    recv_sem=recv_sems.at[d],
                    device_id=(t,),
                    device_id_type=pl.DeviceIdType.MESH,
                ).start()

        @pl.when(s == 1)
        def _():
            xa = x_ref[pl.ds(my * m_per, m_per), :].astype(jnp.bfloat16)
            acc_ref[...] = lax.dot_general(
                xa, wbf_ref[0], dims, preferred_element_type=jnp.float32)

        @pl.when(s > 1)
        def _():
            d = s - 1
            rdma = pltpu.make_async_remote_copy(
                src_ref=x_ref.at[pl.ds(0, m_per), :],
                dst_ref=comm_ref.at[d],
                send_sem=send_sems.at[d],
                recv_sem=recv_sems.at[d],
                device_id=(my,),
                device_id_type=pl.DeviceIdType.MESH,
            )
            rdma.wait_recv()
            xa = comm_ref[d].astype(jnp.bfloat16)
            acc_ref[...] += lax.dot_general(
                xa, wbf_ref[(s - 1) % 2], dims,
                preferred_element_type=jnp.float32)
            rdma.wait_send()

        @pl.when(s < N_DEV)
        def _():
            wbf_ref[s % 2] = w_ref[...].astype(jnp.bfloat16)

        @pl.when(s == N_DEV)
        def _():
            alpha = sx_ref[0] * sw_ref[0]
            out_ref[...] = jnp.maximum(acc_ref[...] * alpha, 0.0)

    grid_spec = pltpu.PrefetchScalarGridSpec(
        num_scalar_prefetch=1,
        grid=(N_DEV + 1,),
        in_specs=[
            pl.BlockSpec((m_total, k_loc), lambda s, wt: (0, 0)),
            pl.BlockSpec((k_total // N_DEV, n_out),
                         lambda s, wt: (wt[s], 0)),
            pl.BlockSpec(memory_space=pltpu.SMEM),
            pl.BlockSpec(memory_space=pltpu.SMEM),
        ],
        out_specs=pl.BlockSpec((m_per, n_out), lambda s, wt: (0, 0)),
        scratch_shapes=[
            pltpu.VMEM((N_DEV, m_per, k_loc), jnp.int8),
            pltpu.VMEM((2, k_total // N_DEV, n_out), jnp.bfloat16),
            pltpu.VMEM((m_per, n_out), jnp.float32),
            pltpu.SemaphoreType.DMA((N_DEV,)),
            pltpu.SemaphoreType.DMA((N_DEV,)),
        ],
    )

    return pl.pallas_call(
        body,
        grid_spec=grid_spec,
        out_shape=jax.ShapeDtypeStruct((m_per, n_out), jnp.float32),
        compiler_params=pltpu.CompilerParams(
            collective_id=0,
            dimension_semantics=("arbitrary",),
            vmem_limit_bytes=64 * 1024 * 1024,
        ),
    )(wtab, x, w_mat, scale_x, scale_w)


# device time: 39930 ns/iter; 1.3119x vs baseline; 1.3119x over previous
import jax
import jax.numpy as jnp
from jax import lax
from jax.experimental import pallas as pl
from jax.experimental.pallas import tpu as pltpu

N_DEV = 16


def kernel(x, w_mat, scale_x, scale_w):
    m_total, k_loc = x.shape
    k_total, n_out = w_mat.shape
    m_per = m_total // N_DEV

    my_pos = lax.axis_index("i")
    steps = jnp.arange(N_DEV, dtype=jnp.int32)
    wtab = jnp.remainder(my_pos.astype(jnp.int32) - steps, N_DEV)
    wtab = jnp.concatenate([wtab, wtab[-1:]])

    def body(wtab_ref, x_ref, w_ref, sx_ref, sw_ref, out_ref,
             comm_ref, wbf_ref, acc_ref, send_sems, recv_sems):
        s = pl.program_id(0)
        my = lax.axis_index("i")
        dims = (((1,), (0,)), ((), ()))

        @pl.when(s == 0)
        def _():
            barrier = pltpu.get_barrier_semaphore()
            for d in range(1, N_DEV):
                t = lax.rem(my + d, N_DEV)
                pl.semaphore_signal(barrier, inc=1, device_id=(t,),
                                    device_id_type=pl.DeviceIdType.MESH)
            pl.semaphore_wait(barrier, N_DEV - 1)

            for d in range(1, N_DEV):
                t = lax.rem(my + d, N_DEV)
                pltpu.make_async_remote_copy(
                    src_ref=x_ref.at[pl.ds(t * m_per, m_per), :],
                    dst_ref=comm_ref.at[d],
                    send_sem=send_sems.at[d],
                    recv_sem=recv_sems.at[d],
                    device_id=(t,),
                    device_id_type=pl.DeviceIdType.MESH,
                ).start()

            comm_ref[0] = x_ref[pl.ds(my * m_per, m_per), :]
            acc_ref[...] = jnp.zeros_like(acc_ref)
            wbf_ref[0] = w_ref[...].astype(jnp.bfloat16)

        @pl.when((s >= 1) & (s < N_DEV))
        def _():
            d = s - 1
            rdma = pltpu.make_async_remote_copy(
                src_ref=x_ref.at[pl.ds(0, m_per), :],
                dst_ref=comm_ref.at[d],
                send_sem=send_sems.at[d],
                recv_sem=recv_sems.at[d],
                device_id=(my,),
                device_id_type=pl.DeviceIdType.MESH,
            )

            @pl.when(d > 0)
            def _():
                rdma.wait_recv()

            xa = comm_ref[d].astype(jnp.bfloat16)
            acc_ref[...] += lax.dot_general(
                xa, wbf_ref[(s - 1) % 2], dims,
                preferred_element_type=jnp.float32)
            wbf_ref[s % 2] = w_ref[...].astype(jnp.bfloat16)

            @pl.when(d > 0)
            def _():
                rdma.wait_send()

        @pl.when(s == N_DEV)
        def _():
            d = N_DEV - 1
            rdma = pltpu.make_async_remote_copy(
                src_ref=x_ref.at[pl.ds(0, m_per), :],
                dst_ref=comm_ref.at[d],
                send_sem=send_sems.at[d],
                recv_sem=recv_sems.at[d],
                device_id=(my,),
                device_id_type=pl.DeviceIdType.MESH,
            )
            rdma.wait_recv()
            xa = comm_ref[d].astype(jnp.bfloat16)
            acc = acc_ref[...] + lax.dot_general(
                xa, wbf_ref[(s - 1) % 2], dims,
                preferred_element_type=jnp.float32)
            rdma.wait_send()
            alpha = sx_ref[0] * sw_ref[0]
            out_ref[...] = jnp.maximum(acc * alpha, 0.0)

    grid_spec = pltpu.PrefetchScalarGridSpec(
        num_scalar_prefetch=1,
        grid=(N_DEV + 1,),
        in_specs=[
            pl.BlockSpec((m_total, k_loc), lambda s, wt: (0, 0)),
            pl.BlockSpec((k_total // N_DEV, n_out),
                         lambda s, wt: (wt[s], 0)),
            pl.BlockSpec(memory_space=pltpu.SMEM),
            pl.BlockSpec(memory_space=pltpu.SMEM),
        ],
        out_specs=pl.BlockSpec((m_per, n_out), lambda s, wt: (0, 0)),
        scratch_shapes=[
            pltpu.VMEM((N_DEV, m_per, k_loc), jnp.int8),
            pltpu.VMEM((2, k_total // N_DEV, n_out), jnp.bfloat16),
            pltpu.VMEM((m_per, n_out), jnp.float32),
            pltpu.SemaphoreType.DMA((N_DEV,)),
            pltpu.SemaphoreType.DMA((N_DEV,)),
        ],
    )

    return pl.pallas_call(
        body,
        grid_spec=grid_spec,
        out_shape=jax.ShapeDtypeStruct((m_per, n_out), jnp.float32),
        compiler_params=pltpu.CompilerParams(
            collective_id=0,
            dimension_semantics=("arbitrary",),
            vmem_limit_bytes=64 * 1024 * 1024,
        ),
    )(wtab, x, w_mat, scale_x, scale_w)
